# baseline (speedup 1.0000x reference)
import functools

import jax
import jax.numpy as jnp
from jax.experimental import pallas as pl

BATCH = 4096
ROW_BLOCK = 256

def _sortable_keys(v):
    i = jax.lax.bitcast_convert_type(v, jnp.int32)
    return jnp.where(i < 0, (-2147483648) - i, i)


def _fused_body(x_ref, w1_ref, b1_ref, w2_ref, b2_ref,
                kw1a_ref, kw1b_ref, kb1_ref, kw2_ref, kb2_ref,
                kw3_ref, kb3_ref, ks_ref,
                khot_ref, k_ref):
    f32 = jnp.float32
    x = x_ref[...]
    jnp_dot = functools.partial(jnp.dot, preferred_element_type=f32)

    h = jnp.maximum(jnp_dot(x, w1_ref[...])
                    + b1_ref[...], 0.0)
    logits = jnp_dot(h, w2_ref[...]) + b2_ref[...]
    qdim = logits.shape[-1]

    a = (jnp_dot(x, kw1a_ref[...])
         + jnp_dot(logits, kw1b_ref[...])
         + kb1_ref[...])
    h1 = jnp.maximum(a, 0.0)
    h2 = jnp.maximum(jnp_dot(h1, kw2_ref[...])
                     + kb2_ref[...], 0.0)
    h2r = h2.astype(jnp.bfloat16).astype(f32)
    w3r = kw3_ref[...].astype(jnp.bfloat16).astype(f32)
    kraw = jnp.sum(h2r * w3r, axis=-1, keepdims=True) + kb3_ref[...]
    k = jax.nn.sigmoid(kraw) * float(qdim)
    k = k * (jax.nn.sigmoid(ks_ref[...]) * 2.0)
    kc = jnp.clip(k, 1.0, float(qdim))
    k_ref[...] = kc

    kf = jnp.ceil(kc)

    key = _sortable_keys(logits)
    h16 = jnp.right_shift(key, 16)
    l16 = jnp.bitwise_and(key, 0xFFFF)

    def count_ge(vals, mid):
        return jnp.sum((vals >= mid).astype(f32), axis=-1, keepdims=True)

    def body1(_, carry):
        lo, hi = carry
        mid = lo + ((hi - lo + 1) >> 1)
        cnt = count_ge(h16, mid)
        take = cnt >= kf
        return jnp.where(take, mid, lo), jnp.where(take, hi, mid - 1)

    lo = jnp.full(kf.shape, -32768, jnp.int32)
    hi = jnp.full(kf.shape, 32767, jnp.int32)
    hstar, _ = jax.lax.fori_loop(0, 16, body1, (lo, hi))

    meq = h16 == hstar
    c_gt_h = jnp.sum((h16 > hstar).astype(f32), axis=-1, keepdims=True)
    k2 = kf - c_gt_h

    def body2(_, carry):
        lo, hi = carry
        mid = lo + ((hi - lo + 1) >> 1)
        cnt = jnp.sum((meq & (l16 >= mid)).astype(f32), axis=-1, keepdims=True)
        take = cnt >= k2
        return jnp.where(take, mid, lo), jnp.where(take, hi, mid - 1)

    lo = jnp.zeros(kf.shape, jnp.int32)
    hi = jnp.full(kf.shape, 65535, jnp.int32)
    lstar, _ = jax.lax.fori_loop(0, 16, body2, (lo, hi))

    gt = (h16 > hstar) | (meq & (l16 > lstar))
    eq = meq & (l16 == lstar)
    c1 = jnp.sum(gt.astype(f32), axis=-1, keepdims=True)
    r = kf - c1

    iota = jax.lax.broadcasted_iota(jnp.int32, logits.shape, 1)

    def body3(_, carry):
        lo, hi = carry
        mid = (lo + hi) >> 1
        cnt = jnp.sum((eq & (iota <= mid)).astype(f32), axis=-1, keepdims=True)
        take = cnt >= r
        return jnp.where(take, lo, mid + 1), jnp.where(take, mid, hi)

    lo = jnp.zeros(kf.shape, jnp.int32)
    hi = jnp.full(kf.shape, qdim - 1, jnp.int32)
    istar, _ = jax.lax.fori_loop(0, 12, body3, (lo, hi))

    khot_ref[...] = (gt | (eq & (iota <= istar))).astype(f32)


@functools.partial(jax.jit, static_argnames=())
def kernel(x, enc_w1, enc_b1, enc_w2, enc_b2,
           kp_w1, kp_b1, kp_w2, kp_b2, kp_w3, kp_b3, k_scale):
    batch, input_dim = x.shape
    n_hdim = enc_w1.shape[1]
    qdim = enc_w2.shape[1]
    rb = ROW_BLOCK if batch % ROW_BLOCK == 0 else batch
    grid = (batch // rb,)

    kp_w1a = kp_w1[:input_dim]
    kp_w1b = kp_w1[input_dim:]

    row_blk = lambda c: pl.BlockSpec((rb, c), lambda i: (i, 0))
    full = lambda a: pl.BlockSpec(a.shape, lambda i: (0,) * a.ndim)

    args = (
        x,
        enc_w1, enc_b1.reshape(1, n_hdim),
        enc_w2, enc_b2.reshape(1, qdim),
        kp_w1a, kp_w1b, kp_b1.reshape(1, n_hdim),
        kp_w2, kp_b2.reshape(1, n_hdim),
        kp_w3.reshape(1, n_hdim), kp_b3.reshape(1, 1),
        k_scale.reshape(1, 1),
    )
    in_specs = [row_blk(input_dim)] + [full(a) for a in args[1:]]

    khot, k = pl.pallas_call(
        _fused_body,
        grid=grid,
        in_specs=in_specs,
        out_specs=[row_blk(qdim), row_blk(1)],
        out_shape=[
            jax.ShapeDtypeStruct((batch, qdim), jnp.float32),
            jax.ShapeDtypeStruct((batch, 1), jnp.float32),
        ],
    )(*args)
    return khot, k


# speedup vs baseline: 1.1859x; 1.1859x over previous
import functools

import jax
import jax.numpy as jnp
from jax.experimental import pallas as pl

BATCH = 4096
ROW_BLOCK = 256

def _fused_body(x_ref, w1_ref, b1_ref, w2_ref, b2_ref,
                kw1a_ref, kw1b_ref, kb1_ref, kw2_ref, kb2_ref,
                kw3_ref, kb3_ref, ks_ref,
                khot_ref, k_ref):
    f32 = jnp.float32
    x = x_ref[...]
    jnp_dot = functools.partial(jnp.dot, preferred_element_type=f32)

    h = jnp.maximum(jnp_dot(x, w1_ref[...])
                    + b1_ref[...], 0.0)
    logits = jnp_dot(h, w2_ref[...]) + b2_ref[...]
    qdim = logits.shape[-1]

    a = (jnp_dot(x, kw1a_ref[...])
         + jnp_dot(logits, kw1b_ref[...])
         + kb1_ref[...])
    h1 = jnp.maximum(a, 0.0)
    h2 = jnp.maximum(jnp_dot(h1, kw2_ref[...])
                     + kb2_ref[...], 0.0)
    h2r = h2.astype(jnp.bfloat16).astype(f32)
    w3r = kw3_ref[...].astype(jnp.bfloat16).astype(f32)
    kraw = jnp.sum(h2r * w3r, axis=-1, keepdims=True) + kb3_ref[...]
    k = jax.nn.sigmoid(kraw) * float(qdim)
    k = k * (jax.nn.sigmoid(ks_ref[...]) * 2.0)
    kc = jnp.clip(k, 1.0, float(qdim))
    k_ref[...] = kc

    kf = jnp.ceil(kc)

    KEY_NEG_INF = -2139095040
    KEY_POS_INF = 2139095040

    def key_to_f32(kint):
        bits = jnp.where(kint >= 0, kint, (-2147483648) - kint)
        return jax.lax.bitcast_convert_type(bits, f32)

    ones_col = jnp.ones((qdim, 1), f32)

    def rowcount(mask):
        return jnp.dot(mask.astype(f32), ones_col,
                       preferred_element_type=f32)

    def count_ge_f(mid_int):
        thr = key_to_f32(mid_int)
        return rowcount(logits >= thr)

    def body1(_, carry):
        lo, hi = carry
        mid = (lo >> 1) + (hi >> 1) + (lo & hi & 1) + ((lo ^ hi) & 1)
        take = count_ge_f(mid) >= kf
        return jnp.where(take, mid, lo), jnp.where(take, hi, mid - 1)

    lo = jnp.full(kf.shape, KEY_NEG_INF, jnp.int32)
    hi = jnp.full(kf.shape, KEY_POS_INF, jnp.int32)
    tkey, _ = jax.lax.fori_loop(0, 32, body1, (lo, hi))
    thr = key_to_f32(tkey)

    gt = logits > thr
    eq = logits == thr
    c1 = rowcount(gt)
    ceq = rowcount(eq)
    r = kf - c1

    iota = jax.lax.broadcasted_iota(jnp.int32, logits.shape, 1)
    qm1 = jnp.full(kf.shape, qdim - 1, jnp.int32)

    def tie_break():
        def body3(_, carry):
            lo3, hi3 = carry
            mid = (lo3 + hi3) >> 1
            cnt = rowcount(eq & (iota <= mid))
            take = cnt >= r
            return jnp.where(take, lo3, mid + 1), jnp.where(take, mid, hi3)

        lo3 = jnp.zeros(kf.shape, jnp.int32)
        out, _ = jax.lax.fori_loop(0, 12, body3, (lo3, qm1))
        return out

    istar = jax.lax.cond(jnp.all(ceq == r), lambda: qm1, tie_break)
    khot_ref[...] = (gt | (eq & (iota <= istar))).astype(f32)


@functools.partial(jax.jit, static_argnames=())
def kernel(x, enc_w1, enc_b1, enc_w2, enc_b2,
           kp_w1, kp_b1, kp_w2, kp_b2, kp_w3, kp_b3, k_scale):
    batch, input_dim = x.shape
    n_hdim = enc_w1.shape[1]
    qdim = enc_w2.shape[1]
    rb = ROW_BLOCK if batch % ROW_BLOCK == 0 else batch
    grid = (batch // rb,)

    kp_w1a = kp_w1[:input_dim]
    kp_w1b = kp_w1[input_dim:]

    row_blk = lambda c: pl.BlockSpec((rb, c), lambda i: (i, 0))
    full = lambda a: pl.BlockSpec(a.shape, lambda i: (0,) * a.ndim)

    args = (
        x,
        enc_w1, enc_b1.reshape(1, n_hdim),
        enc_w2, enc_b2.reshape(1, qdim),
        kp_w1a, kp_w1b, kp_b1.reshape(1, n_hdim),
        kp_w2, kp_b2.reshape(1, n_hdim),
        kp_w3.reshape(1, n_hdim), kp_b3.reshape(1, 1),
        k_scale.reshape(1, 1),
    )
    in_specs = [row_blk(input_dim)] + [full(a) for a in args[1:]]

    khot, k = pl.pallas_call(
        _fused_body,
        grid=grid,
        in_specs=in_specs,
        out_specs=[row_blk(qdim), row_blk(1)],
        out_shape=[
            jax.ShapeDtypeStruct((batch, qdim), jnp.float32),
            jax.ShapeDtypeStruct((batch, 1), jnp.float32),
        ],
    )(*args)
    return khot, k
